# baseline (speedup 1.0000x reference)
import functools

import jax
import jax.numpy as jnp
from jax.experimental import pallas as pl
from jax.experimental.pallas import tpu as pltpu

_NEIGHBOURING_K = 128
_N_SAMPLING = 4096
_RADIUS0 = 0.4
_BN_EPS = 1e-5
_BN_SCALE = 1.0


def _bis(values, indices):
    return jax.vmap(lambda v, i: v[i])(values, indices)


_FPS_R = 8
_FPS_C = 1024


def _fps_kernel(pos_ref, out_ref, dists_ref):
    px = pos_ref[0, 0]
    py = pos_ref[0, 1]
    pz = pos_ref[0, 2]
    lin = (jax.lax.broadcasted_iota(jnp.int32, (_FPS_R, _FPS_C), 0) * _FPS_C
           + jax.lax.broadcasted_iota(jnp.int32, (_FPS_R, _FPS_C), 1))

    def dist_to(sx, sy, sz):
        dx = px - sx
        dy = py - sy
        dz = pz - sz
        return (dx * dx + dz * dz) + dy * dy

    out_ref[0, 0, 0] = 0
    sx0 = px[0:1, 0:1]
    sy0 = py[0:1, 0:1]
    sz0 = pz[0:1, 0:1]
    dists_ref[...] = dist_to(sx0, sy0, sz0)

    def body(i, carry):
        sx, sy, sz = carry
        d = jnp.minimum(dists_ref[...], dist_to(sx, sy, sz))
        dists_ref[...] = d
        m = jnp.max(d)
        cand = jnp.where(d == m, lin, jnp.int32(8192))
        nxt = jnp.min(cand)
        out_ref[0, 0, i] = nxt
        sel = lin == nxt
        nsx = jnp.sum(jnp.where(sel, px, 0.0), keepdims=True)
        nsy = jnp.sum(jnp.where(sel, py, 0.0), keepdims=True)
        nsz = jnp.sum(jnp.where(sel, pz, 0.0), keepdims=True)
        return (nsx, nsy, nsz)

    jax.lax.fori_loop(1, _N_SAMPLING, body, (sx0, sy0, sz0))


def _fps(pos, n_sampling):
    bz = pos.shape[0]
    pos_t = jnp.transpose(pos, (0, 2, 1)).reshape(bz, 3, _FPS_R, _FPS_C)
    out = pl.pallas_call(
        _fps_kernel,
        grid=(bz,),
        in_specs=[pl.BlockSpec((1, 3, _FPS_R, _FPS_C), lambda b: (b, 0, 0, 0))],
        out_specs=pl.BlockSpec((1, 1, n_sampling), lambda b: (b, 0, 0),
                               memory_space=pltpu.SMEM),
        out_shape=jax.ShapeDtypeStruct((bz, 1, n_sampling), jnp.int32),
        scratch_shapes=[pltpu.VMEM((_FPS_R, _FPS_C), jnp.float32)],
    )(pos_t)
    return out[:, 0, :]


def _mlp(feat, layers):
    for (w, b, g, bt) in layers:
        feat = jnp.einsum('...i,io->...o', feat, w) + b
        feat = feat / jnp.sqrt(1.0 + _BN_EPS) * g + bt
        feat = jax.nn.relu(feat)
    return feat


_KNN_R = 16


def _cmpx(key, idx, j, down):
    lane = jax.lax.broadcasted_iota(jnp.int32, key.shape, key.ndim - 1)
    bit = (lane & j) != 0
    pk = jnp.where(bit, pltpu.roll(key, j, key.ndim - 1),
                   pltpu.roll(key, 128 - j, key.ndim - 1))
    pi = jnp.where(bit, pltpu.roll(idx, j, idx.ndim - 1),
                   pltpu.roll(idx, 128 - j, idx.ndim - 1))
    less = (pk < key) | ((pk == key) & (pi < idx))
    take = less ^ bit
    if down is not None:
        take = take ^ down
    return jnp.where(take, pk, key), jnp.where(take, pi, idx)


def _knn_kernel(pos_ref, spos_ref, idx_ref, bias_ref):
    px = pos_ref[0, 0]
    py = pos_ref[0, 1]
    pz = pos_ref[0, 2]
    sx = spos_ref[0, :, 0].reshape(_KNN_R, 1, 1)
    sy = spos_ref[0, :, 1].reshape(_KNN_R, 1, 1)
    sz = spos_ref[0, :, 2].reshape(_KNN_R, 1, 1)
    dx = px[None] - sx
    dy = py[None] - sy
    dz = pz[None] - sz
    sq = (dx * dx + dy * dy) + dz * dz
    key = jnp.maximum(sq, 1e-12)
    idx = (jax.lax.broadcasted_iota(jnp.int32, key.shape, 1) * 128
           + jax.lax.broadcasted_iota(jnp.int32, key.shape, 2))

    desc0 = jax.lax.broadcasted_iota(jnp.int32, key.shape, 1) >= 32

    def outer(p, carry):
        k, i = carry
        kk = jnp.int32(1) << p
        lane = jax.lax.broadcasted_iota(jnp.int32, k.shape, 2)
        down = ((lane & kk) != 0) ^ desc0

        def inner(t, c2):
            return _cmpx(c2[0], c2[1], kk >> (t + 1), down)

        return jax.lax.fori_loop(0, p, inner, (k, i))

    key, idx = jax.lax.fori_loop(1, 8, outer, (key, idx))

    c = 64
    while c > 1:
        half = c // 2
        ak, ai = key[:, :half], idx[:, :half]
        bk, bi = key[:, half:c], idx[:, half:c]
        lessb = (bk < ak) | ((bk == ak) & (bi < ai))
        key = jnp.where(lessb, bk, ak)
        idx = jnp.where(lessb, bi, ai)
        if half > 1:
            desc = (jax.lax.broadcasted_iota(jnp.int32, key.shape, 1)
                    >= half // 2)
        else:
            desc = jnp.zeros(key.shape, jnp.bool_)

        def clean(t, c2, _desc=desc):
            return _cmpx(c2[0], c2[1], jnp.int32(64) >> t, _desc)

        key, idx = jax.lax.fori_loop(0, 7, clean, (key, idx))
        c = half

    idx_ref[0] = idx[:, 0, :]
    ppd = jnp.sqrt(key[:, 0, :])
    bias_ref[0] = jnp.where(ppd <= _RADIUS0, 0.0, -1e8).astype(jnp.float32)


def _knn(pos, sampled_pos):
    bz, n, _ = pos.shape
    ns = sampled_pos.shape[1]
    pos_r = jnp.transpose(pos, (0, 2, 1)).reshape(bz, 3, 64, 128)
    grid = (bz, ns // _KNN_R)
    idx, bias = pl.pallas_call(
        _knn_kernel,
        grid=grid,
        in_specs=[
            pl.BlockSpec((1, 3, 64, 128), lambda b, t: (b, 0, 0, 0)),
            pl.BlockSpec((1, _KNN_R, 3), lambda b, t: (b, t, 0)),
        ],
        out_specs=[
            pl.BlockSpec((1, _KNN_R, 128), lambda b, t: (b, t, 0)),
            pl.BlockSpec((1, _KNN_R, 128), lambda b, t: (b, t, 0)),
        ],
        out_shape=[
            jax.ShapeDtypeStruct((bz, ns, 128), jnp.int32),
            jax.ShapeDtypeStruct((bz, ns, 128), jnp.float32),
        ],
    )(pos_r, sampled_pos)
    return idx, bias


_MLP0_T = 128


def _mlp0_kernel(feat_ref, w1_ref, p1_ref, w2_ref, p2_ref, w3_ref, p3_ref,
                 bias_ref, e_ref, out_ref):
    inv = 1.0 / jnp.sqrt(1.0 + _BN_EPS)
    a = feat_ref[0]
    h = jnp.dot(a, w1_ref[...], preferred_element_type=jnp.float32)
    h = jax.nn.relu(h * inv * p1_ref[0] + p1_ref[1])
    h = jnp.dot(h, w2_ref[...], preferred_element_type=jnp.float32)
    h = jax.nn.relu(h * inv * p2_ref[0] + p2_ref[1])
    h = jnp.dot(h, w3_ref[...], preferred_element_type=jnp.float32)
    h = jax.nn.relu(h * inv * p3_ref[0] + p3_ref[1])
    h = h + jnp.dot(bias_ref[0], e_ref[...],
                    preferred_element_type=jnp.float32)
    m = jnp.maximum(h, pltpu.roll(h, 1024 - 128, 1))
    m = jnp.maximum(m, pltpu.roll(m, 1024 - 256, 1))
    m = jnp.maximum(m, pltpu.roll(m, 1024 - 512, 1))
    m128 = m[:, :128]
    out_ref[0] = jnp.max(m128.reshape(_MLP0_T, 16, 128), axis=1)


def _mlp0(grouped_feat, bias, layers):
    bz, ns, k, _ = grouped_feat.shape
    inv = 1.0 / jnp.sqrt(1.0 + _BN_EPS)
    eye8 = jnp.eye(8, dtype=jnp.float32)
    (w1, b1, g1, bt1), (w2, b2, g2, bt2), (w3, b3, g3, bt3) = layers
    w1p = jnp.pad(w1, ((0, 2), (0, 0)))
    W1 = jnp.kron(eye8, w1p)
    W2 = jnp.kron(eye8, w2)
    W3 = jnp.kron(eye8, w3)
    p1 = jnp.stack([jnp.tile(g1, 8), jnp.tile(b1 * inv * g1 + bt1, 8)])
    p2 = jnp.stack([jnp.tile(g2, 8), jnp.tile(b2 * inv * g2 + bt2, 8)])
    p3 = jnp.stack([jnp.tile(g3, 8), jnp.tile(b3 * inv * g3 + bt3, 8)])
    E = jnp.kron(eye8, jnp.ones((1, 128), jnp.float32))
    gf8 = jnp.pad(grouped_feat, ((0, 0), (0, 0), (0, 0), (0, 2))
                  ).reshape(bz, ns * k // 8, 64)
    bias8 = bias.reshape(bz, ns * k // 8, 8)
    rows = _MLP0_T * 16
    grid = (bz, ns // _MLP0_T)
    out = pl.pallas_call(
        _mlp0_kernel,
        grid=grid,
        in_specs=[
            pl.BlockSpec((1, rows, 64), lambda b, t: (b, t, 0)),
            pl.BlockSpec((64, 512), lambda b, t: (0, 0)),
            pl.BlockSpec((2, 512), lambda b, t: (0, 0)),
            pl.BlockSpec((512, 512), lambda b, t: (0, 0)),
            pl.BlockSpec((2, 512), lambda b, t: (0, 0)),
            pl.BlockSpec((512, 1024), lambda b, t: (0, 0)),
            pl.BlockSpec((2, 1024), lambda b, t: (0, 0)),
            pl.BlockSpec((1, rows, 8), lambda b, t: (b, t, 0)),
            pl.BlockSpec((8, 1024), lambda b, t: (0, 0)),
        ],
        out_specs=pl.BlockSpec((1, _MLP0_T, 128), lambda b, t: (b, t, 0)),
        out_shape=jax.ShapeDtypeStruct((bz, ns, 128), jnp.float32),
    )(gf8, W1, p1, W2, p2, W3, p3, bias8, E)
    return out


def _stage2_kernel(feat_ref, w1_ref, b1_ref, w2_ref, b2_ref, w3_ref, b3_ref,
                   out_ref):
    t = pl.program_id(1)
    f = feat_ref[0]
    inv = 1.0 / jnp.sqrt(1.0 + _BN_EPS)
    h = jnp.dot(f, w1_ref[...], preferred_element_type=jnp.float32)
    h = jax.nn.relu(h * inv * b1_ref[0] + b1_ref[1])
    h = jnp.dot(h, w2_ref[...], preferred_element_type=jnp.float32)
    h = jax.nn.relu(h * inv * b2_ref[0] + b2_ref[1])
    h = jnp.dot(h, w3_ref[...], preferred_element_type=jnp.float32)
    h = jax.nn.relu(h * inv * b3_ref[0] + b3_ref[1])
    m = jnp.max(h, axis=0, keepdims=True)[None]

    @pl.when(t == 0)
    def _init():
        out_ref[...] = m

    @pl.when(t != 0)
    def _acc():
        out_ref[...] = jnp.maximum(out_ref[...], m)


def _stage2(feat, layers):
    bz, n, _ = feat.shape
    inv = 1.0 / jnp.sqrt(1.0 + _BN_EPS)
    packed = []
    for (w, b, g, bt) in layers:
        scale = g
        shift = b * inv * g + bt
        packed.append((w, jnp.stack([scale, shift])))
    (w1, p1), (w2, p2), (w3, p3) = packed
    T = 1024
    grid = (bz, n // T)
    out = pl.pallas_call(
        _stage2_kernel,
        grid=grid,
        in_specs=[
            pl.BlockSpec((1, T, 131), lambda b, t: (b, t, 0)),
            pl.BlockSpec((131, 128), lambda b, t: (0, 0)),
            pl.BlockSpec((2, 128), lambda b, t: (0, 0)),
            pl.BlockSpec((128, 128), lambda b, t: (0, 0)),
            pl.BlockSpec((2, 128), lambda b, t: (0, 0)),
            pl.BlockSpec((128, 1024), lambda b, t: (0, 0)),
            pl.BlockSpec((2, 1024), lambda b, t: (0, 0)),
        ],
        out_specs=pl.BlockSpec((1, 1, 1024), lambda b, t: (b, 0, 0)),
        out_shape=jax.ShapeDtypeStruct((bz, 1, 1024), jnp.float32),
    )(feat, w1, p1, w2, p2, w3, p3)
    return out[:, 0, :]


def kernel(x, pos, params):
    bz = pos.shape[0]
    fps_idx = _fps(pos[:, :, :3], _N_SAMPLING)
    sampled_pos = _bis(pos, fps_idx)
    topk_idx, bias = _knn(pos, sampled_pos)
    grouped_pos = _bis(pos, topk_idx) - sampled_pos[:, :, None, :]
    grouped_feat = jnp.concatenate([grouped_pos, _bis(x, topk_idx)], axis=-1)
    feat0 = _mlp0(grouped_feat, bias, params[0])

    gfeat = jnp.concatenate([sampled_pos, feat0], axis=-1)
    global_x = _stage2(gfeat, params[1])

    rt_pp_feat = jnp.swapaxes(feat0, -1, -2)
    return rt_pp_feat, global_x, topk_idx


# speedup vs baseline: 1.1684x; 1.1684x over previous
import jax
import jax.numpy as jnp
from jax.experimental import pallas as pl
from jax.experimental.pallas import tpu as pltpu

_NEIGHBOURING_K = 128
_N_SAMPLING = 4096
_RADIUS0 = 0.4
_BN_EPS = 1e-5


def _bis(values, indices):
    return jax.vmap(lambda v, i: v[i])(values, indices)


_FPS_R = 8
_FPS_C = 1024


def _fps_kernel(pos_ref, out_ref, dists_ref):
    px = pos_ref[0, 0]
    py = pos_ref[0, 1]
    pz = pos_ref[0, 2]
    lin = (jax.lax.broadcasted_iota(jnp.int32, (_FPS_R, _FPS_C), 0) * _FPS_C
           + jax.lax.broadcasted_iota(jnp.int32, (_FPS_R, _FPS_C), 1))

    def dist_to(sx, sy, sz):
        dx = px - sx
        dy = py - sy
        dz = pz - sz
        return (dx * dx + dz * dz) + dy * dy

    out_ref[0, 0, 0] = 0
    sx0 = px[0:1, 0:1]
    sy0 = py[0:1, 0:1]
    sz0 = pz[0:1, 0:1]
    dists_ref[...] = dist_to(sx0, sy0, sz0)

    def body(i, carry):
        sx, sy, sz = carry
        d = jnp.minimum(dists_ref[...], dist_to(sx, sy, sz))
        dists_ref[...] = d
        m = jnp.max(d)
        cand = jnp.where(d == m, lin, jnp.int32(8192))
        nxt = jnp.min(cand)
        out_ref[0, 0, i] = nxt
        sel = lin == nxt
        nsx = jnp.sum(jnp.where(sel, px, 0.0), keepdims=True)
        nsy = jnp.sum(jnp.where(sel, py, 0.0), keepdims=True)
        nsz = jnp.sum(jnp.where(sel, pz, 0.0), keepdims=True)
        return (nsx, nsy, nsz)

    jax.lax.fori_loop(1, _N_SAMPLING, body, (sx0, sy0, sz0))


def _fps(pos, n_sampling):
    bz = pos.shape[0]
    pos_t = jnp.transpose(pos, (0, 2, 1)).reshape(bz, 3, _FPS_R, _FPS_C)
    out = pl.pallas_call(
        _fps_kernel,
        grid=(bz,),
        in_specs=[pl.BlockSpec((1, 3, _FPS_R, _FPS_C), lambda b: (b, 0, 0, 0))],
        out_specs=pl.BlockSpec((1, 1, n_sampling), lambda b: (b, 0, 0),
                               memory_space=pltpu.SMEM),
        out_shape=jax.ShapeDtypeStruct((bz, 1, n_sampling), jnp.int32),
        scratch_shapes=[pltpu.VMEM((_FPS_R, _FPS_C), jnp.float32)],
    )(pos_t)
    return out[:, 0, :]


def _mlp(feat, layers):
    for (w, b, g, bt) in layers:
        feat = jnp.einsum('...i,io->...o', feat, w) + b
        feat = feat / jnp.sqrt(1.0 + _BN_EPS) * g + bt
        feat = jax.nn.relu(feat)
    return feat


_KNN_R = 16


def _cmpx(key, idx, j, down):
    lane = jax.lax.broadcasted_iota(jnp.int32, key.shape, key.ndim - 1)
    bit = (lane & j) != 0
    pk = jnp.where(bit, pltpu.roll(key, j, key.ndim - 1),
                   pltpu.roll(key, 128 - j, key.ndim - 1))
    pi = jnp.where(bit, pltpu.roll(idx, j, idx.ndim - 1),
                   pltpu.roll(idx, 128 - j, idx.ndim - 1))
    less = (pk < key) | ((pk == key) & (pi < idx))
    take = less ^ bit
    if down is not None:
        take = take ^ down
    return jnp.where(take, pk, key), jnp.where(take, pi, idx)


def _knn_kernel(pos_ref, spos_ref, idx_ref, bias_ref):
    px = pos_ref[0, 0]
    py = pos_ref[0, 1]
    pz = pos_ref[0, 2]
    sx = spos_ref[0, :, 0].reshape(_KNN_R, 1, 1)
    sy = spos_ref[0, :, 1].reshape(_KNN_R, 1, 1)
    sz = spos_ref[0, :, 2].reshape(_KNN_R, 1, 1)
    dx = px[None] - sx
    dy = py[None] - sy
    dz = pz[None] - sz
    sq = (dx * dx + dy * dy) + dz * dz
    key = jnp.maximum(sq, 1e-12)
    idx = (jax.lax.broadcasted_iota(jnp.int32, key.shape, 1) * 128
           + jax.lax.broadcasted_iota(jnp.int32, key.shape, 2))

    desc0 = jax.lax.broadcasted_iota(jnp.int32, key.shape, 1) >= 32

    def outer(p, carry):
        k, i = carry
        kk = jnp.int32(1) << p
        lane = jax.lax.broadcasted_iota(jnp.int32, k.shape, 2)
        down = ((lane & kk) != 0) ^ desc0

        def inner(t, c2):
            return _cmpx(c2[0], c2[1], kk >> (t + 1), down)

        return jax.lax.fori_loop(0, p, inner, (k, i))

    key, idx = jax.lax.fori_loop(1, 8, outer, (key, idx))

    c = 64
    while c > 1:
        half = c // 2
        ak, ai = key[:, :half], idx[:, :half]
        bk, bi = key[:, half:c], idx[:, half:c]
        lessb = (bk < ak) | ((bk == ak) & (bi < ai))
        key = jnp.where(lessb, bk, ak)
        idx = jnp.where(lessb, bi, ai)
        if half > 1:
            desc = (jax.lax.broadcasted_iota(jnp.int32, key.shape, 1)
                    >= half // 2)
        else:
            desc = jnp.zeros(key.shape, jnp.bool_)

        def clean(t, c2, _desc=desc):
            return _cmpx(c2[0], c2[1], jnp.int32(64) >> t, _desc)

        key, idx = jax.lax.fori_loop(0, 7, clean, (key, idx))
        c = half

    idx_ref[0] = idx[:, 0, :]
    ppd = jnp.sqrt(key[:, 0, :])
    bias_ref[0] = jnp.where(ppd <= _RADIUS0, 0.0, -1e8).astype(jnp.float32)


def _knn(pos, sampled_pos):
    bz, n, _ = pos.shape
    ns = sampled_pos.shape[1]
    pos_r = jnp.transpose(pos, (0, 2, 1)).reshape(bz, 3, 64, 128)
    grid = (bz, ns // _KNN_R)
    idx, bias = pl.pallas_call(
        _knn_kernel,
        grid=grid,
        in_specs=[
            pl.BlockSpec((1, 3, 64, 128), lambda b, t: (b, 0, 0, 0)),
            pl.BlockSpec((1, _KNN_R, 3), lambda b, t: (b, t, 0)),
        ],
        out_specs=[
            pl.BlockSpec((1, _KNN_R, 128), lambda b, t: (b, t, 0)),
            pl.BlockSpec((1, _KNN_R, 128), lambda b, t: (b, t, 0)),
        ],
        out_shape=[
            jax.ShapeDtypeStruct((bz, ns, 128), jnp.int32),
            jax.ShapeDtypeStruct((bz, ns, 128), jnp.float32),
        ],
    )(pos_r, sampled_pos)
    return idx, bias


def _stage2_kernel(feat_ref, w1_ref, b1_ref, w2_ref, b2_ref, w3_ref, b3_ref,
                   out_ref):
    t = pl.program_id(1)
    f = feat_ref[0]
    inv = 1.0 / jnp.sqrt(1.0 + _BN_EPS)
    h = jnp.dot(f, w1_ref[...], preferred_element_type=jnp.float32)
    h = jax.nn.relu(h * inv * b1_ref[0] + b1_ref[1])
    h = jnp.dot(h, w2_ref[...], preferred_element_type=jnp.float32)
    h = jax.nn.relu(h * inv * b2_ref[0] + b2_ref[1])
    h = jnp.dot(h, w3_ref[...], preferred_element_type=jnp.float32)
    h = jax.nn.relu(h * inv * b3_ref[0] + b3_ref[1])
    m = jnp.max(h, axis=0, keepdims=True)[None]

    @pl.when(t == 0)
    def _init():
        out_ref[...] = m

    @pl.when(t != 0)
    def _acc():
        out_ref[...] = jnp.maximum(out_ref[...], m)


def _stage2(feat, layers):
    bz, n, _ = feat.shape
    inv = 1.0 / jnp.sqrt(1.0 + _BN_EPS)
    packed = []
    for (w, b, g, bt) in layers:
        scale = g
        shift = b * inv * g + bt
        packed.append((w, jnp.stack([scale, shift])))
    (w1, p1), (w2, p2), (w3, p3) = packed
    T = 1024
    grid = (bz, n // T)
    out = pl.pallas_call(
        _stage2_kernel,
        grid=grid,
        in_specs=[
            pl.BlockSpec((1, T, 131), lambda b, t: (b, t, 0)),
            pl.BlockSpec((131, 128), lambda b, t: (0, 0)),
            pl.BlockSpec((2, 128), lambda b, t: (0, 0)),
            pl.BlockSpec((128, 128), lambda b, t: (0, 0)),
            pl.BlockSpec((2, 128), lambda b, t: (0, 0)),
            pl.BlockSpec((128, 1024), lambda b, t: (0, 0)),
            pl.BlockSpec((2, 1024), lambda b, t: (0, 0)),
        ],
        out_specs=pl.BlockSpec((1, 1, 1024), lambda b, t: (b, 0, 0)),
        out_shape=jax.ShapeDtypeStruct((bz, 1, 1024), jnp.float32),
    )(feat, w1, p1, w2, p2, w3, p3)
    return out[:, 0, :]


def kernel(x, pos, params):
    bz = pos.shape[0]
    fps_idx = _fps(pos[:, :, :3], _N_SAMPLING)
    sampled_pos = _bis(pos, fps_idx)
    topk_idx, bias = _knn(pos, sampled_pos)
    grouped_pos = _bis(pos, topk_idx) - sampled_pos[:, :, None, :]
    grouped_feat = jnp.concatenate([grouped_pos, _bis(x, topk_idx)], axis=-1)
    gf = _mlp(grouped_feat, params[0])
    gf = gf + bias[..., None]
    feat0 = jnp.max(gf, axis=2)

    gfeat = jnp.concatenate([sampled_pos, feat0], axis=-1)
    global_x = _stage2(gfeat, params[1])

    rt_pp_feat = jnp.swapaxes(feat0, -1, -2)
    return rt_pp_feat, global_x, topk_idx


# speedup vs baseline: 1.1880x; 1.0168x over previous
import jax
import jax.numpy as jnp
from jax.experimental import pallas as pl
from jax.experimental.pallas import tpu as pltpu

_NEIGHBOURING_K = 128
_N_SAMPLING = 4096
_RADIUS0 = 0.4
_BN_EPS = 1e-5


def _bis(values, indices):
    return jax.vmap(lambda v, i: v[i])(values, indices)


_FPS_R = 8
_FPS_C = 1024


def _fps_kernel(pos_ref, out_ref, dists_ref):
    px = pos_ref[0, 0]
    py = pos_ref[0, 1]
    pz = pos_ref[0, 2]
    lin = (jax.lax.broadcasted_iota(jnp.int32, (_FPS_R, _FPS_C), 0) * _FPS_C
           + jax.lax.broadcasted_iota(jnp.int32, (_FPS_R, _FPS_C), 1))

    def dist_to(sx, sy, sz):
        dx = px - sx
        dy = py - sy
        dz = pz - sz
        return (dx * dx + dz * dz) + dy * dy

    out_ref[0, 0, 0] = 0
    sx0 = px[0:1, 0:1]
    sy0 = py[0:1, 0:1]
    sz0 = pz[0:1, 0:1]
    dists_ref[...] = dist_to(sx0, sy0, sz0)

    def body(i, carry):
        sx, sy, sz = carry
        d = jnp.minimum(dists_ref[...], dist_to(sx, sy, sz))
        dists_ref[...] = d
        m = jnp.max(d)
        cand = jnp.where(d == m, lin, jnp.int32(8192))
        nxt = jnp.min(cand)
        out_ref[0, 0, i] = nxt
        sel = lin == nxt
        nsx = jnp.sum(jnp.where(sel, px, 0.0), keepdims=True)
        nsy = jnp.sum(jnp.where(sel, py, 0.0), keepdims=True)
        nsz = jnp.sum(jnp.where(sel, pz, 0.0), keepdims=True)
        return (nsx, nsy, nsz)

    jax.lax.fori_loop(1, _N_SAMPLING, body, (sx0, sy0, sz0))


def _fps(pos, n_sampling):
    bz = pos.shape[0]
    pos_t = jnp.transpose(pos, (0, 2, 1)).reshape(bz, 3, _FPS_R, _FPS_C)
    out = pl.pallas_call(
        _fps_kernel,
        grid=(bz,),
        in_specs=[pl.BlockSpec((1, 3, _FPS_R, _FPS_C), lambda b: (b, 0, 0, 0))],
        out_specs=pl.BlockSpec((1, 1, n_sampling), lambda b: (b, 0, 0),
                               memory_space=pltpu.SMEM),
        out_shape=jax.ShapeDtypeStruct((bz, 1, n_sampling), jnp.int32),
        scratch_shapes=[pltpu.VMEM((_FPS_R, _FPS_C), jnp.float32)],
    )(pos_t)
    return out[:, 0, :]


def _mlp(feat, layers):
    for (w, b, g, bt) in layers:
        feat = jnp.einsum('...i,io->...o', feat, w) + b
        feat = feat / jnp.sqrt(1.0 + _BN_EPS) * g + bt
        feat = jax.nn.relu(feat)
    return feat


_KNN_R = 32


def _cmpx(key, idx, j, down):
    lane = jax.lax.broadcasted_iota(jnp.int32, key.shape, key.ndim - 1)
    bit = (lane & j) != 0
    pk = jnp.where(bit, pltpu.roll(key, j, key.ndim - 1),
                   pltpu.roll(key, 128 - j, key.ndim - 1))
    pi = jnp.where(bit, pltpu.roll(idx, j, idx.ndim - 1),
                   pltpu.roll(idx, 128 - j, idx.ndim - 1))
    less = (pk < key) | ((pk == key) & (pi < idx))
    take = less ^ bit
    if down is not None:
        take = take ^ down
    return jnp.where(take, pk, key), jnp.where(take, pi, idx)


def _knn_kernel(pos_ref, spos_ref, idx_ref, bias_ref):
    px = pos_ref[0, 0]
    py = pos_ref[0, 1]
    pz = pos_ref[0, 2]
    sx = spos_ref[0, :, 0].reshape(_KNN_R, 1, 1)
    sy = spos_ref[0, :, 1].reshape(_KNN_R, 1, 1)
    sz = spos_ref[0, :, 2].reshape(_KNN_R, 1, 1)
    dx = px[None] - sx
    dy = py[None] - sy
    dz = pz[None] - sz
    sq = (dx * dx + dy * dy) + dz * dz
    key = jnp.maximum(sq, 1e-12)
    idx = (jax.lax.broadcasted_iota(jnp.int32, key.shape, 1) * 128
           + jax.lax.broadcasted_iota(jnp.int32, key.shape, 2))

    desc0 = jax.lax.broadcasted_iota(jnp.int32, key.shape, 1) >= 32

    def outer(p, carry):
        k, i = carry
        kk = jnp.int32(1) << p
        lane = jax.lax.broadcasted_iota(jnp.int32, k.shape, 2)
        down = ((lane & kk) != 0) ^ desc0

        def inner(t, c2):
            return _cmpx(c2[0], c2[1], kk >> (t + 1), down)

        return jax.lax.fori_loop(0, p, inner, (k, i))

    key, idx = jax.lax.fori_loop(1, 8, outer, (key, idx))

    c = 64
    while c > 1:
        half = c // 2
        ak, ai = key[:, :half], idx[:, :half]
        bk, bi = key[:, half:c], idx[:, half:c]
        lessb = (bk < ak) | ((bk == ak) & (bi < ai))
        key = jnp.where(lessb, bk, ak)
        idx = jnp.where(lessb, bi, ai)
        if half > 1:
            desc = (jax.lax.broadcasted_iota(jnp.int32, key.shape, 1)
                    >= half // 2)
        else:
            desc = jnp.zeros(key.shape, jnp.bool_)

        def clean(t, c2, _desc=desc):
            return _cmpx(c2[0], c2[1], jnp.int32(64) >> t, _desc)

        key, idx = jax.lax.fori_loop(0, 7, clean, (key, idx))
        c = half

    idx_ref[0] = idx[:, 0, :]
    ppd = jnp.sqrt(key[:, 0, :])
    bias_ref[0] = jnp.where(ppd <= _RADIUS0, 0.0, -1e8).astype(jnp.float32)


def _knn(pos, sampled_pos):
    bz, n, _ = pos.shape
    ns = sampled_pos.shape[1]
    pos_r = jnp.transpose(pos, (0, 2, 1)).reshape(bz, 3, 64, 128)
    grid = (bz, ns // _KNN_R)
    idx, bias = pl.pallas_call(
        _knn_kernel,
        grid=grid,
        in_specs=[
            pl.BlockSpec((1, 3, 64, 128), lambda b, t: (b, 0, 0, 0)),
            pl.BlockSpec((1, _KNN_R, 3), lambda b, t: (b, t, 0)),
        ],
        out_specs=[
            pl.BlockSpec((1, _KNN_R, 128), lambda b, t: (b, t, 0)),
            pl.BlockSpec((1, _KNN_R, 128), lambda b, t: (b, t, 0)),
        ],
        out_shape=[
            jax.ShapeDtypeStruct((bz, ns, 128), jnp.int32),
            jax.ShapeDtypeStruct((bz, ns, 128), jnp.float32),
        ],
    )(pos_r, sampled_pos)
    return idx, bias


def _stage2_kernel(feat_ref, w1_ref, b1_ref, w2_ref, b2_ref, w3_ref, b3_ref,
                   out_ref):
    t = pl.program_id(1)
    f = feat_ref[0]
    inv = 1.0 / jnp.sqrt(1.0 + _BN_EPS)
    h = jnp.dot(f, w1_ref[...], preferred_element_type=jnp.float32)
    h = jax.nn.relu(h * inv * b1_ref[0] + b1_ref[1])
    h = jnp.dot(h, w2_ref[...], preferred_element_type=jnp.float32)
    h = jax.nn.relu(h * inv * b2_ref[0] + b2_ref[1])
    h = jnp.dot(h, w3_ref[...], preferred_element_type=jnp.float32)
    h = jax.nn.relu(h * inv * b3_ref[0] + b3_ref[1])
    m = jnp.max(h, axis=0, keepdims=True)[None]

    @pl.when(t == 0)
    def _init():
        out_ref[...] = m

    @pl.when(t != 0)
    def _acc():
        out_ref[...] = jnp.maximum(out_ref[...], m)


def _stage2(feat, layers):
    bz, n, _ = feat.shape
    inv = 1.0 / jnp.sqrt(1.0 + _BN_EPS)
    packed = []
    for (w, b, g, bt) in layers:
        scale = g
        shift = b * inv * g + bt
        packed.append((w, jnp.stack([scale, shift])))
    (w1, p1), (w2, p2), (w3, p3) = packed
    T = 1024
    grid = (bz, n // T)
    out = pl.pallas_call(
        _stage2_kernel,
        grid=grid,
        in_specs=[
            pl.BlockSpec((1, T, 131), lambda b, t: (b, t, 0)),
            pl.BlockSpec((131, 128), lambda b, t: (0, 0)),
            pl.BlockSpec((2, 128), lambda b, t: (0, 0)),
            pl.BlockSpec((128, 128), lambda b, t: (0, 0)),
            pl.BlockSpec((2, 128), lambda b, t: (0, 0)),
            pl.BlockSpec((128, 1024), lambda b, t: (0, 0)),
            pl.BlockSpec((2, 1024), lambda b, t: (0, 0)),
        ],
        out_specs=pl.BlockSpec((1, 1, 1024), lambda b, t: (b, 0, 0)),
        out_shape=jax.ShapeDtypeStruct((bz, 1, 1024), jnp.float32),
    )(feat, w1, p1, w2, p2, w3, p3)
    return out[:, 0, :]


def kernel(x, pos, params):
    bz = pos.shape[0]
    fps_idx = _fps(pos[:, :, :3], _N_SAMPLING)
    sampled_pos = _bis(pos, fps_idx)
    topk_idx, bias = _knn(pos, sampled_pos)
    grouped_pos = _bis(pos, topk_idx) - sampled_pos[:, :, None, :]
    grouped_feat = jnp.concatenate([grouped_pos, _bis(x, topk_idx)], axis=-1)
    gf = _mlp(grouped_feat, params[0])
    gf = gf + bias[..., None]
    feat0 = jnp.max(gf, axis=2)

    gfeat = jnp.concatenate([sampled_pos, feat0], axis=-1)
    global_x = _stage2(gfeat, params[1])

    rt_pp_feat = jnp.swapaxes(feat0, -1, -2)
    return rt_pp_feat, global_x, topk_idx


# speedup vs baseline: 1.1961x; 1.0068x over previous
import jax
import jax.numpy as jnp
from jax.experimental import pallas as pl
from jax.experimental.pallas import tpu as pltpu

_NEIGHBOURING_K = 128
_N_SAMPLING = 4096
_RADIUS0 = 0.4
_BN_EPS = 1e-5


def _bis(values, indices):
    return jax.vmap(lambda v, i: v[i])(values, indices)


_FPS_R = 8
_FPS_C = 1024


def _fps_kernel(pos_ref, out_ref, dists_ref):
    px = pos_ref[0, 0]
    py = pos_ref[0, 1]
    pz = pos_ref[0, 2]
    lin = (jax.lax.broadcasted_iota(jnp.int32, (_FPS_R, _FPS_C), 0) * _FPS_C
           + jax.lax.broadcasted_iota(jnp.int32, (_FPS_R, _FPS_C), 1))

    def dist_to(sx, sy, sz):
        dx = px - sx
        dy = py - sy
        dz = pz - sz
        return (dx * dx + dz * dz) + dy * dy

    out_ref[0, 0, 0] = 0
    sx0 = px[0:1, 0:1]
    sy0 = py[0:1, 0:1]
    sz0 = pz[0:1, 0:1]
    dists_ref[...] = dist_to(sx0, sy0, sz0)

    def body(i, carry):
        sx, sy, sz = carry
        d = jnp.minimum(dists_ref[...], dist_to(sx, sy, sz))
        dists_ref[...] = d
        m = jnp.max(d)
        cand = jnp.where(d == m, lin, jnp.int32(8192))
        nxt = jnp.min(cand)
        out_ref[0, 0, i] = nxt
        sel = lin == nxt
        nsx = jnp.sum(jnp.where(sel, px, 0.0), keepdims=True)
        nsy = jnp.sum(jnp.where(sel, py, 0.0), keepdims=True)
        nsz = jnp.sum(jnp.where(sel, pz, 0.0), keepdims=True)
        return (nsx, nsy, nsz)

    jax.lax.fori_loop(1, _N_SAMPLING, body, (sx0, sy0, sz0))


def _fps(pos, n_sampling):
    bz = pos.shape[0]
    pos_t = jnp.transpose(pos, (0, 2, 1)).reshape(bz, 3, _FPS_R, _FPS_C)
    out = pl.pallas_call(
        _fps_kernel,
        grid=(bz,),
        in_specs=[pl.BlockSpec((1, 3, _FPS_R, _FPS_C), lambda b: (b, 0, 0, 0))],
        out_specs=pl.BlockSpec((1, 1, n_sampling), lambda b: (b, 0, 0),
                               memory_space=pltpu.SMEM),
        out_shape=jax.ShapeDtypeStruct((bz, 1, n_sampling), jnp.int32),
        scratch_shapes=[pltpu.VMEM((_FPS_R, _FPS_C), jnp.float32)],
    )(pos_t)
    return out[:, 0, :]


def _mlp(feat, layers):
    for (w, b, g, bt) in layers:
        feat = jnp.einsum('...i,io->...o', feat, w) + b
        feat = feat / jnp.sqrt(1.0 + _BN_EPS) * g + bt
        feat = jax.nn.relu(feat)
    return feat


_KNN_R = 64


def _cmpx(key, idx, j, down):
    lane = jax.lax.broadcasted_iota(jnp.int32, key.shape, key.ndim - 1)
    bit = (lane & j) != 0
    pk = jnp.where(bit, pltpu.roll(key, j, key.ndim - 1),
                   pltpu.roll(key, 128 - j, key.ndim - 1))
    pi = jnp.where(bit, pltpu.roll(idx, j, idx.ndim - 1),
                   pltpu.roll(idx, 128 - j, idx.ndim - 1))
    less = (pk < key) | ((pk == key) & (pi < idx))
    take = less ^ bit
    if down is not None:
        take = take ^ down
    return jnp.where(take, pk, key), jnp.where(take, pi, idx)


def _knn_kernel(pos_ref, spos_ref, idx_ref, bias_ref):
    px = pos_ref[0, 0]
    py = pos_ref[0, 1]
    pz = pos_ref[0, 2]
    sx = spos_ref[0, :, 0].reshape(_KNN_R, 1, 1)
    sy = spos_ref[0, :, 1].reshape(_KNN_R, 1, 1)
    sz = spos_ref[0, :, 2].reshape(_KNN_R, 1, 1)
    dx = px[None] - sx
    dy = py[None] - sy
    dz = pz[None] - sz
    sq = (dx * dx + dy * dy) + dz * dz
    key = jnp.maximum(sq, 1e-12)
    idx = (jax.lax.broadcasted_iota(jnp.int32, key.shape, 1) * 128
           + jax.lax.broadcasted_iota(jnp.int32, key.shape, 2))

    desc0 = jax.lax.broadcasted_iota(jnp.int32, key.shape, 1) >= 32

    def outer(p, carry):
        k, i = carry
        kk = jnp.int32(1) << p
        lane = jax.lax.broadcasted_iota(jnp.int32, k.shape, 2)
        down = ((lane & kk) != 0) ^ desc0

        def inner(t, c2):
            return _cmpx(c2[0], c2[1], kk >> (t + 1), down)

        return jax.lax.fori_loop(0, p, inner, (k, i))

    key, idx = jax.lax.fori_loop(1, 8, outer, (key, idx))

    c = 64
    while c > 1:
        half = c // 2
        ak, ai = key[:, :half], idx[:, :half]
        bk, bi = key[:, half:c], idx[:, half:c]
        lessb = (bk < ak) | ((bk == ak) & (bi < ai))
        key = jnp.where(lessb, bk, ak)
        idx = jnp.where(lessb, bi, ai)
        if half > 1:
            desc = (jax.lax.broadcasted_iota(jnp.int32, key.shape, 1)
                    >= half // 2)
        else:
            desc = jnp.zeros(key.shape, jnp.bool_)

        def clean(t, c2, _desc=desc):
            return _cmpx(c2[0], c2[1], jnp.int32(64) >> t, _desc)

        key, idx = jax.lax.fori_loop(0, 7, clean, (key, idx))
        c = half

    idx_ref[0] = idx[:, 0, :]
    ppd = jnp.sqrt(key[:, 0, :])
    bias_ref[0] = jnp.where(ppd <= _RADIUS0, 0.0, -1e8).astype(jnp.float32)


def _knn(pos, sampled_pos):
    bz, n, _ = pos.shape
    ns = sampled_pos.shape[1]
    pos_r = jnp.transpose(pos, (0, 2, 1)).reshape(bz, 3, 64, 128)
    grid = (bz, ns // _KNN_R)
    idx, bias = pl.pallas_call(
        _knn_kernel,
        grid=grid,
        in_specs=[
            pl.BlockSpec((1, 3, 64, 128), lambda b, t: (b, 0, 0, 0)),
            pl.BlockSpec((1, _KNN_R, 3), lambda b, t: (b, t, 0)),
        ],
        out_specs=[
            pl.BlockSpec((1, _KNN_R, 128), lambda b, t: (b, t, 0)),
            pl.BlockSpec((1, _KNN_R, 128), lambda b, t: (b, t, 0)),
        ],
        out_shape=[
            jax.ShapeDtypeStruct((bz, ns, 128), jnp.int32),
            jax.ShapeDtypeStruct((bz, ns, 128), jnp.float32),
        ],
    )(pos_r, sampled_pos)
    return idx, bias


def _stage2_kernel(feat_ref, w1_ref, b1_ref, w2_ref, b2_ref, w3_ref, b3_ref,
                   out_ref):
    t = pl.program_id(1)
    f = feat_ref[0]
    inv = 1.0 / jnp.sqrt(1.0 + _BN_EPS)
    h = jnp.dot(f, w1_ref[...], preferred_element_type=jnp.float32)
    h = jax.nn.relu(h * inv * b1_ref[0] + b1_ref[1])
    h = jnp.dot(h, w2_ref[...], preferred_element_type=jnp.float32)
    h = jax.nn.relu(h * inv * b2_ref[0] + b2_ref[1])
    h = jnp.dot(h, w3_ref[...], preferred_element_type=jnp.float32)
    h = jax.nn.relu(h * inv * b3_ref[0] + b3_ref[1])
    m = jnp.max(h, axis=0, keepdims=True)[None]

    @pl.when(t == 0)
    def _init():
        out_ref[...] = m

    @pl.when(t != 0)
    def _acc():
        out_ref[...] = jnp.maximum(out_ref[...], m)


def _stage2(feat, layers):
    bz, n, _ = feat.shape
    inv = 1.0 / jnp.sqrt(1.0 + _BN_EPS)
    packed = []
    for (w, b, g, bt) in layers:
        scale = g
        shift = b * inv * g + bt
        packed.append((w, jnp.stack([scale, shift])))
    (w1, p1), (w2, p2), (w3, p3) = packed
    T = 1024
    grid = (bz, n // T)
    out = pl.pallas_call(
        _stage2_kernel,
        grid=grid,
        in_specs=[
            pl.BlockSpec((1, T, 131), lambda b, t: (b, t, 0)),
            pl.BlockSpec((131, 128), lambda b, t: (0, 0)),
            pl.BlockSpec((2, 128), lambda b, t: (0, 0)),
            pl.BlockSpec((128, 128), lambda b, t: (0, 0)),
            pl.BlockSpec((2, 128), lambda b, t: (0, 0)),
            pl.BlockSpec((128, 1024), lambda b, t: (0, 0)),
            pl.BlockSpec((2, 1024), lambda b, t: (0, 0)),
        ],
        out_specs=pl.BlockSpec((1, 1, 1024), lambda b, t: (b, 0, 0)),
        out_shape=jax.ShapeDtypeStruct((bz, 1, 1024), jnp.float32),
    )(feat, w1, p1, w2, p2, w3, p3)
    return out[:, 0, :]


def kernel(x, pos, params):
    bz = pos.shape[0]
    fps_idx = _fps(pos[:, :, :3], _N_SAMPLING)
    sampled_pos = _bis(pos, fps_idx)
    topk_idx, bias = _knn(pos, sampled_pos)
    grouped_pos = _bis(pos, topk_idx) - sampled_pos[:, :, None, :]
    grouped_feat = jnp.concatenate([grouped_pos, _bis(x, topk_idx)], axis=-1)
    gf = _mlp(grouped_feat, params[0])
    gf = gf + bias[..., None]
    feat0 = jnp.max(gf, axis=2)

    gfeat = jnp.concatenate([sampled_pos, feat0], axis=-1)
    global_x = _stage2(gfeat, params[1])

    rt_pp_feat = jnp.swapaxes(feat0, -1, -2)
    return rt_pp_feat, global_x, topk_idx
